# baseline (speedup 1.0000x reference)
import functools

import jax
import jax.numpy as jnp
from jax import lax
from jax.experimental import pallas as pl
from jax.experimental.pallas import tpu as pltpu
from jax.experimental.pallas import tpu_sc as plsc

EPS = 0.001


def _prep_body(x_pos_ref, slope_ref, y_bias_ref, xpad_ref, slopec_ref, icept_ref):
    x = x_pos_ref[...]
    s = slope_ref[...]
    yb = y_bias_ref[...]
    fb, k = x.shape

    slope_c = jnp.log(1.0 + jnp.exp(-jnp.abs(s))) + jnp.maximum(s, 0.0) + EPS

    a = x[:, :, None]
    b = x[:, None, :]
    ii = lax.broadcasted_iota(jnp.int32, (fb, k, k), 1)
    jj = lax.broadcasted_iota(jnp.int32, (fb, k, k), 2)
    prec = (a < b) | ((a == b) & (ii < jj))
    rank = jnp.sum(prec.astype(jnp.int32), axis=1)
    onehot = (rank[:, :, None] == jj).astype(jnp.float32)
    xs = jnp.sum(x[:, :, None] * onehot, axis=1)

    d = xs - jnp.concatenate([xs[:, :1], xs[:, :-1]], axis=1)
    t = d * slope_c[:, :k]
    col = lax.broadcasted_iota(jnp.int32, (fb, k), 1)
    t = jnp.where(col == 0, jnp.broadcast_to(xs[:, :1] + yb, (fb, k)), t)
    tri = (lax.broadcasted_iota(jnp.int32, (k, k), 0)
           <= lax.broadcasted_iota(jnp.int32, (k, k), 1)).astype(jnp.float32)
    y_pos = jax.lax.dot_general(t, tri, (((1,), (0,)), ((), ())),
                                preferred_element_type=jnp.float32,
                                precision=jax.lax.Precision.HIGHEST)

    inf_col = jnp.full((fb, 1), jnp.inf, jnp.float32)
    xpad_ref[...] = jnp.concatenate([xs, inf_col], axis=1)
    slopec_ref[...] = slope_c
    x_sel = jnp.concatenate([xs[:, :1], xs], axis=1)
    y_sel = jnp.concatenate([y_pos[:, :1], y_pos], axis=1)
    icept_ref[...] = y_sel - slope_c * x_sel


def _prep_tables(x_pos, slope, y_bias, interpret=False):
    f, k = x_pos.shape
    fb = 128
    grid = (f // fb,)
    out_shape = [jax.ShapeDtypeStruct((f, k + 1), jnp.float32)] * 3
    return pl.pallas_call(
        _prep_body,
        grid=grid,
        in_specs=[
            pl.BlockSpec((fb, k), lambda i: (i, 0)),
            pl.BlockSpec((fb, k + 1), lambda i: (i, 0)),
            pl.BlockSpec((fb, 1), lambda i: (i, 0)),
        ],
        out_specs=[pl.BlockSpec((fb, k + 1), lambda i: (i, 0))] * 3,
        out_shape=out_shape,
        interpret=interpret,
    )(x_pos, slope, y_bias)


def _sc_spline(v_hbm, xtab_hbm, stab_hbm, itab_hbm, out_hbm, slope_hbm,
               xtab, stab, itab, vin0, vin1, vout0, vout1, vslp0, vslp1,
               isem0, isem1, osem0, osem1,
               *, B, F, K, CG, RG, FW, RW, RCH, UNROLL):
    ns1 = K + 1
    tw = FW * ns1
    nc = 2
    wid = lax.axis_index("s") * nc + lax.axis_index("c")
    cg = wid % CG
    rg = wid // CG
    row0 = rg * RW
    col0 = cg * FW

    pltpu.sync_copy(xtab_hbm.at[pl.ds(cg * tw, tw)], xtab)
    pltpu.sync_copy(stab_hbm.at[pl.ds(cg * tw, tw)], stab)
    pltpu.sync_copy(itab_hbm.at[pl.ds(cg * tw, tw)], itab)

    lane = lax.iota(jnp.int32, 16)
    nch = RW // RCH
    nvec = RCH * FW // 16
    cmask = FW // 16 - 1
    cshift = (FW // 16).bit_length() - 1

    def in_slab(ch):
        return v_hbm.at[pl.ds(row0 + ch * RCH, RCH), pl.ds(col0, FW)]

    def compute(vin, vout, vslp):
        @plsc.parallel_loop(0, nvec, unroll=UNROLL)
        def _vec(t):
            c = t & cmask
            r = lax.shift_right_logical(t, cshift)
            v = vin[r, pl.ds(c * 16, 16)]
            p = (lane + c * 16) * ns1
            for step, off in ((16, 15), (8, 7), (4, 3), (2, 1), (1, 0)):
                xm = plsc.load_gather(xtab, [p + off])
                p = jnp.where(xm < v, p + step, p)
            xm = plsc.load_gather(xtab, [p])
            p = jnp.where(xm < v, p + 1, p)
            sl = plsc.load_gather(stab, [p])
            ic = plsc.load_gather(itab, [p])
            vout[r, pl.ds(c * 16, 16)] = ic + v * sl
            vslp[r, pl.ds(c * 16, 16)] = sl

    def start_out(ch, vout, vslp, osem):
        pltpu.async_copy(vout, out_hbm.at[pl.ds(row0 + ch * RCH, RCH),
                                          pl.ds(col0, FW)], osem)
        pltpu.async_copy(vslp, slope_hbm.at[pl.ds(row0 + ch * RCH, RCH),
                                            pl.ds(col0, FW)], osem)
    def wait_out(ch, vout, vslp, osem):
        pltpu.make_async_copy(vout, out_hbm.at[pl.ds(row0 + ch * RCH, RCH),
                                               pl.ds(col0, FW)], osem).wait()
        pltpu.make_async_copy(vslp, slope_hbm.at[pl.ds(row0 + ch * RCH, RCH),
                                                 pl.ds(col0, FW)], osem).wait()

    ng = nch // 2
    pltpu.async_copy(in_slab(0), vin0, isem0)

    def chunk_pair(g, _):
        c0 = 2 * g
        c1 = 2 * g + 1
        pltpu.async_copy(in_slab(c1), vin1, isem1)
        pltpu.make_async_copy(in_slab(c0), vin0, isem0).wait()

        @pl.when(g > 0)
        def _w0():
            wait_out(c0 - 2, vout0, vslp0, osem0)
        compute(vin0, vout0, vslp0)
        start_out(c0, vout0, vslp0, osem0)

        @pl.when(g + 1 < ng)
        def _s0():
            pltpu.async_copy(in_slab(c0 + 2), vin0, isem0)
        pltpu.make_async_copy(in_slab(c1), vin1, isem1).wait()

        @pl.when(g > 0)
        def _w1():
            wait_out(c1 - 2, vout1, vslp1, osem1)
        compute(vin1, vout1, vslp1)
        start_out(c1, vout1, vslp1, osem1)
        return _

    lax.fori_loop(0, ng, chunk_pair, 0)
    wait_out(nch - 2, vout0, vslp0, osem0)
    wait_out(nch - 1, vout1, vslp1, osem1)


def _sc_main(inputs, xflat, sflat, iflat):
    B, F = inputs.shape
    K = 32
    CG, RG = 4, 8
    FW = F // CG
    RW = B // RG
    RCH = 16
    UNROLL = 8
    tw = FW * (K + 1)

    mesh = plsc.VectorSubcoreMesh(core_axis_name="c", subcore_axis_name="s")
    body = functools.partial(
        _sc_spline, B=B, F=F, K=K, CG=CG, RG=RG, FW=FW, RW=RW, RCH=RCH,
        UNROLL=UNROLL)
    buf = lambda: pltpu.VMEM((RCH, FW), jnp.float32)
    return pl.kernel(
        body,
        out_type=(jax.ShapeDtypeStruct((B, F), jnp.float32),
                  jax.ShapeDtypeStruct((B, F), jnp.float32)),
        mesh=mesh,
        compiler_params=pltpu.CompilerParams(needs_layout_passes=False),
        scratch_types=[
            pltpu.VMEM((tw,), jnp.float32),
            pltpu.VMEM((tw,), jnp.float32),
            pltpu.VMEM((tw,), jnp.float32),
            buf(), buf(),
            buf(), buf(),
            buf(), buf(),
            pltpu.SemaphoreType.DMA,
            pltpu.SemaphoreType.DMA,
            pltpu.SemaphoreType.DMA,
            pltpu.SemaphoreType.DMA,
        ],
    )(inputs, xflat, sflat, iflat)


def kernel(inputs, x_pos, slope, y_bias):
    xpad, slope_c, icept = _prep_tables(x_pos, slope, y_bias)
    out, slope_sel = _sc_main(
        inputs, xpad.reshape(-1), slope_c.reshape(-1), icept.reshape(-1))
    return out, slope_sel


# speedup vs baseline: 2.0010x; 2.0010x over previous
import functools

import jax
import jax.numpy as jnp
from jax import lax
from jax.experimental import pallas as pl
from jax.experimental.pallas import tpu as pltpu
from jax.experimental.pallas import tpu_sc as plsc

EPS = 0.001


def _prep_body(x_pos_ref, slope_ref, y_bias_ref, xpad_ref, slopec_ref, icept_ref):
    x = x_pos_ref[...]
    s = slope_ref[...]
    yb = y_bias_ref[...]
    fb, k = x.shape

    slope_c = jnp.log(1.0 + jnp.exp(-jnp.abs(s))) + jnp.maximum(s, 0.0) + EPS

    a = x[:, :, None]
    b = x[:, None, :]
    ii = lax.broadcasted_iota(jnp.int32, (fb, k, k), 1)
    jj = lax.broadcasted_iota(jnp.int32, (fb, k, k), 2)
    prec = (a < b) | ((a == b) & (ii < jj))
    rank = jnp.sum(prec.astype(jnp.int32), axis=1)
    onehot = (rank[:, :, None] == jj).astype(jnp.float32)
    xs = jnp.sum(x[:, :, None] * onehot, axis=1)

    d = xs - jnp.concatenate([xs[:, :1], xs[:, :-1]], axis=1)
    t = d * slope_c[:, :k]
    col = lax.broadcasted_iota(jnp.int32, (fb, k), 1)
    t = jnp.where(col == 0, jnp.broadcast_to(xs[:, :1] + yb, (fb, k)), t)
    tri = (lax.broadcasted_iota(jnp.int32, (k, k), 0)
           <= lax.broadcasted_iota(jnp.int32, (k, k), 1)).astype(jnp.float32)
    y_pos = jax.lax.dot_general(t, tri, (((1,), (0,)), ((), ())),
                                preferred_element_type=jnp.float32,
                                precision=jax.lax.Precision.HIGHEST)

    inf_col = jnp.full((fb, 1), jnp.inf, jnp.float32)
    xpad_ref[...] = jnp.concatenate([xs, inf_col], axis=1)
    slopec_ref[...] = slope_c
    x_sel = jnp.concatenate([xs[:, :1], xs], axis=1)
    y_sel = jnp.concatenate([y_pos[:, :1], y_pos], axis=1)
    icept_ref[...] = y_sel - slope_c * x_sel


def _prep_tables(x_pos, slope, y_bias, interpret=False):
    f, k = x_pos.shape
    fb = 128
    grid = (f // fb,)
    out_shape = [jax.ShapeDtypeStruct((f, k + 1), jnp.float32)] * 3
    return pl.pallas_call(
        _prep_body,
        grid=grid,
        in_specs=[
            pl.BlockSpec((fb, k), lambda i: (i, 0)),
            pl.BlockSpec((fb, k + 1), lambda i: (i, 0)),
            pl.BlockSpec((fb, 1), lambda i: (i, 0)),
        ],
        out_specs=[pl.BlockSpec((fb, k + 1), lambda i: (i, 0))] * 3,
        out_shape=out_shape,
        interpret=interpret,
    )(x_pos, slope, y_bias)


def _sc_spline(v_hbm, xtab_hbm, stab_hbm, itab_hbm, out_hbm, slope_hbm,
               xtab, stab, itab, vin0, vin1, vout0, vout1, vslp0, vslp1,
               isem0, isem1, osem0, osem1,
               *, B, F, K, CG, RG, FW, RW, RCH, UNROLL):
    ns1 = K + 1
    tw = FW * ns1
    nc = 2
    wid = lax.axis_index("s") * nc + lax.axis_index("c")
    cg = wid % CG
    rg = wid // CG
    row0 = rg * RW
    col0 = cg * FW

    pltpu.sync_copy(xtab_hbm.at[pl.ds(cg * tw, tw)], xtab)
    pltpu.sync_copy(stab_hbm.at[pl.ds(cg * tw, tw)], stab)
    pltpu.sync_copy(itab_hbm.at[pl.ds(cg * tw, tw)], itab)

    lane = lax.iota(jnp.int32, 16)
    nch = RW // RCH
    nvec = RCH * FW // 16
    cmask = FW // 16 - 1
    cshift = (FW // 16).bit_length() - 1

    def in_slab(ch):
        return v_hbm.at[pl.ds(row0 + ch * RCH, RCH), pl.ds(col0, FW)]

    def compute(vin, vout, vslp):
        @plsc.parallel_loop(0, nvec, unroll=UNROLL)
        def _vec(t):
            c = t & cmask
            r = lax.shift_right_logical(t, cshift)
            v = vin[r, pl.ds(c * 16, 16)]
            p = lane + c * 16
            for step, off in ((16, 15), (8, 7), (4, 3), (2, 1), (1, 0)):
                xm = plsc.load_gather(xtab, [p + off * FW])
                p = jnp.where(xm < v, p + step * FW, p)
            xm = plsc.load_gather(xtab, [p])
            p = jnp.where(xm < v, p + FW, p)
            sl = plsc.load_gather(stab, [p])
            ic = plsc.load_gather(itab, [p])
            vout[r, pl.ds(c * 16, 16)] = ic + v * sl
            vslp[r, pl.ds(c * 16, 16)] = sl

    def start_out(ch, vout, vslp, osem):
        pltpu.async_copy(vout, out_hbm.at[pl.ds(row0 + ch * RCH, RCH),
                                          pl.ds(col0, FW)], osem)
        pltpu.async_copy(vslp, slope_hbm.at[pl.ds(row0 + ch * RCH, RCH),
                                            pl.ds(col0, FW)], osem)
    def wait_out(ch, vout, vslp, osem):
        pltpu.make_async_copy(vout, out_hbm.at[pl.ds(row0 + ch * RCH, RCH),
                                               pl.ds(col0, FW)], osem).wait()
        pltpu.make_async_copy(vslp, slope_hbm.at[pl.ds(row0 + ch * RCH, RCH),
                                                 pl.ds(col0, FW)], osem).wait()

    ng = nch // 2
    pltpu.async_copy(in_slab(0), vin0, isem0)

    def chunk_pair(g, _):
        c0 = 2 * g
        c1 = 2 * g + 1
        pltpu.async_copy(in_slab(c1), vin1, isem1)
        pltpu.make_async_copy(in_slab(c0), vin0, isem0).wait()

        @pl.when(g > 0)
        def _w0():
            wait_out(c0 - 2, vout0, vslp0, osem0)
        compute(vin0, vout0, vslp0)
        start_out(c0, vout0, vslp0, osem0)

        @pl.when(g + 1 < ng)
        def _s0():
            pltpu.async_copy(in_slab(c0 + 2), vin0, isem0)
        pltpu.make_async_copy(in_slab(c1), vin1, isem1).wait()

        @pl.when(g > 0)
        def _w1():
            wait_out(c1 - 2, vout1, vslp1, osem1)
        compute(vin1, vout1, vslp1)
        start_out(c1, vout1, vslp1, osem1)
        return _

    lax.fori_loop(0, ng, chunk_pair, 0)
    wait_out(nch - 2, vout0, vslp0, osem0)
    wait_out(nch - 1, vout1, vslp1, osem1)


def _sc_main(inputs, xflat, sflat, iflat):
    B, F = inputs.shape
    K = 32
    CG, RG = 4, 8
    FW = F // CG
    RW = B // RG
    RCH = 16
    UNROLL = 4
    tw = FW * (K + 1)

    mesh = plsc.VectorSubcoreMesh(core_axis_name="c", subcore_axis_name="s")
    body = functools.partial(
        _sc_spline, B=B, F=F, K=K, CG=CG, RG=RG, FW=FW, RW=RW, RCH=RCH,
        UNROLL=UNROLL)
    buf = lambda: pltpu.VMEM((RCH, FW), jnp.float32)
    return pl.kernel(
        body,
        out_type=(jax.ShapeDtypeStruct((B, F), jnp.float32),
                  jax.ShapeDtypeStruct((B, F), jnp.float32)),
        mesh=mesh,
        compiler_params=pltpu.CompilerParams(needs_layout_passes=False),
        scratch_types=[
            pltpu.VMEM((tw,), jnp.float32),
            pltpu.VMEM((tw,), jnp.float32),
            pltpu.VMEM((tw,), jnp.float32),
            buf(), buf(),
            buf(), buf(),
            buf(), buf(),
            pltpu.SemaphoreType.DMA,
            pltpu.SemaphoreType.DMA,
            pltpu.SemaphoreType.DMA,
            pltpu.SemaphoreType.DMA,
        ],
    )(inputs, xflat, sflat, iflat)


def kernel(inputs, x_pos, slope, y_bias):
    xpad, slope_c, icept = _prep_tables(x_pos, slope, y_bias)
    f, ns1 = xpad.shape
    cg, fw = 4, f // 4

    def regroup(t):
        return t.T.reshape(ns1, cg, fw).transpose(1, 0, 2).reshape(-1)

    out, slope_sel = _sc_main(
        inputs, regroup(xpad), regroup(slope_c), regroup(icept))
    return out, slope_sel
